# Initial kernel scaffold; baseline (speedup 1.0000x reference)
import functools

import jax
import jax.numpy as jnp
from jax import lax
from jax.experimental import pallas as pl
from jax.experimental.pallas import tpu as pltpu
from jax.experimental.pallas import tpu_sc as plsc

_N = 1 << 16
_L = 16
_NC = 2
_NS = 16
_ROWS_PER_W = _N // (_NC * _NS)
_CHUNKS = _ROWS_PER_W // _L


def _sc_body(lut_hbm, out_hbm, lut_v, out_v, sem):
    wid = lax.axis_index("s") * _NC + lax.axis_index("c")
    base = wid * _ROWS_PER_W

    pltpu.sync_copy(lut_hbm, lut_v)

    lane = jnp.arange(_L, dtype=jnp.int32)
    zeros = jnp.zeros((_L,), jnp.int32)
    ones = jnp.ones((_L,), jnp.int32)

    def chunk(k, carry):
        j = k * _L + lane
        i = base + j
        t = i * (i + 1)
        e = lax.shift_right_logical(t, 6) & 511
        c0 = plsc.load_gather(lut_v, [e, zeros])
        c1 = plsc.load_gather(lut_v, [e, ones])
        s = lax.shift_left(t & 0x8000, 16)
        c0 = plsc.bitcast(plsc.bitcast(c0, jnp.int32) ^ s, jnp.float32)
        plsc.store_scatter(out_v, [j, zeros], c0)
        plsc.store_scatter(out_v, [j, ones], c1)
        return carry

    lax.fori_loop(0, _CHUNKS, chunk, 0, unroll=4)

    pltpu.sync_copy(out_v, out_hbm.at[pl.ds(base, _ROWS_PER_W)])


@jax.jit
def kernel(lut):
    mesh = plsc.VectorSubcoreMesh(core_axis_name="c", subcore_axis_name="s")
    run = functools.partial(
        pl.kernel,
        out_type=jax.ShapeDtypeStruct((_N, 2), jnp.float32),
        mesh=mesh,
        scratch_types=[
            pltpu.VMEM((512, 2), jnp.float32),
            pltpu.VMEM((_ROWS_PER_W, 2), jnp.float32),
            pltpu.SemaphoreType.DMA,
        ],
    )(_sc_body)
    return run(lut.astype(jnp.float32))


# speedup vs baseline: 4.7874x; 4.7874x over previous
import functools

import jax
import jax.numpy as jnp
from jax import lax
from jax.experimental import pallas as pl
from jax.experimental.pallas import tpu as pltpu
from jax.experimental.pallas import tpu_sc as plsc

_N = 1 << 16
_L = 16
_NC = 2
_NS = 16
_ROWS_PER_W = _N // (_NC * _NS)
_CHUNKS = _ROWS_PER_W // _L


def _sc_body(lut_hbm, out_hbm, lut_v, out_v, sem):
    wid = lax.axis_index("s") * _NC + lax.axis_index("c")
    base = wid * _ROWS_PER_W

    pltpu.sync_copy(lut_hbm, lut_v)

    lane = jnp.arange(_L, dtype=jnp.int32)

    @plsc.parallel_loop(0, _ROWS_PER_W, step=_L, unroll=4)
    def chunk(k):
        j = k + lane
        i = base + j
        t = i * (i + 1)
        e = lax.shift_right_logical(t, jnp.int32(5)) & jnp.int32(1022)
        c0 = plsc.load_gather(lut_v, [e])
        c1 = plsc.load_gather(lut_v, [e | jnp.int32(1)])
        s = lax.shift_left(t & jnp.int32(0x8000), jnp.int32(16))
        c0 = plsc.bitcast(plsc.bitcast(c0, jnp.int32) ^ s, jnp.float32)
        jf = jnp.int32(2) * j
        plsc.store_scatter(out_v, [jf], c0)
        plsc.store_scatter(out_v, [jf | jnp.int32(1)], c1)

    pltpu.sync_copy(out_v, out_hbm.at[pl.ds(base * 2, 2 * _ROWS_PER_W)])


@jax.jit
def kernel(lut):
    with jax.enable_x64(False):
        mesh = plsc.VectorSubcoreMesh(core_axis_name="c", subcore_axis_name="s")
        run = functools.partial(
            pl.kernel,
            out_type=jax.ShapeDtypeStruct((2 * _N,), jnp.float32),
            mesh=mesh,
            scratch_types=[
                pltpu.VMEM((1024,), jnp.float32),
                pltpu.VMEM((2 * _ROWS_PER_W,), jnp.float32),
                pltpu.SemaphoreType.DMA,
            ],
            compiler_params=pltpu.CompilerParams(needs_layout_passes=False),
        )(_sc_body)
        flat = run(lut.astype(jnp.float32).reshape(1024))
        return flat.reshape(_N, 2)


# speedup vs baseline: 17.0465x; 3.5607x over previous
import functools

import jax
import jax.numpy as jnp
from jax import lax
from jax.experimental import pallas as pl
from jax.experimental.pallas import tpu as pltpu
from jax.experimental.pallas import tpu_sc as plsc

_N = 1 << 16
_L = 16
_NC = 2
_NS = 16
_ROWS_PER_W = _N // (_NC * _NS)


def _sc_body(lut_hbm, out_hbm, lut_v, out_v, sem):
    wid = lax.axis_index("s") * _NC + lax.axis_index("c")
    base = wid * _ROWS_PER_W

    pltpu.sync_copy(lut_hbm, lut_v)

    lane = jnp.arange(_L, dtype=jnp.int32)

    @plsc.parallel_loop(0, _ROWS_PER_W, step=_L, unroll=4)
    def chunk(k):
        j = k + lane
        i = base + j
        t = i * (i + 1)
        e = lax.shift_right_logical(t, jnp.int32(5)) & jnp.int32(1022)
        c0 = plsc.load_gather(lut_v, [e])
        c1 = plsc.load_gather(lut_v, [e | jnp.int32(1)])
        s = lax.shift_left(t & jnp.int32(0x8000), jnp.int32(16))
        c0 = plsc.bitcast(plsc.bitcast(c0, jnp.int32) ^ s, jnp.float32)
        off = (lax.shift_right_logical(k, jnp.int32(7)) << jnp.int32(8)) | (
            k & jnp.int32(127))
        out_v[pl.ds(off, _L)] = c0
        out_v[pl.ds(off + jnp.int32(128), _L)] = c1

    pltpu.sync_copy(out_v, out_hbm.at[pl.ds(base * 2, 2 * _ROWS_PER_W)])


@jax.jit
def kernel(lut):
    with jax.enable_x64(False):
        mesh = plsc.VectorSubcoreMesh(core_axis_name="c", subcore_axis_name="s")
        run = functools.partial(
            pl.kernel,
            out_type=jax.ShapeDtypeStruct((2 * _N,), jnp.float32),
            mesh=mesh,
            scratch_types=[
                pltpu.VMEM((1024,), jnp.float32),
                pltpu.VMEM((2 * _ROWS_PER_W,), jnp.float32),
                pltpu.SemaphoreType.DMA,
            ],
            compiler_params=pltpu.CompilerParams(needs_layout_passes=False),
        )(_sc_body)
        flat = run(lut.astype(jnp.float32).reshape(1024))
        return flat.reshape(512, 2, 128).transpose(0, 2, 1).reshape(_N, 2)


# speedup vs baseline: 17.1125x; 1.0039x over previous
import functools

import jax
import jax.numpy as jnp
from jax import lax
from jax.experimental import pallas as pl
from jax.experimental.pallas import tpu as pltpu
from jax.experimental.pallas import tpu_sc as plsc

_N = 1 << 16
_L = 16
_NC = 2
_NS = 16
_ROWS_PER_W = _N // (_NC * _NS)


def _sc_body(lut_hbm, out_hbm, lut_v, out_v, sem):
    wid = lax.axis_index("s") * _NC + lax.axis_index("c")
    base = wid * _ROWS_PER_W

    pltpu.sync_copy(lut_hbm, lut_v)

    lane = jnp.arange(_L, dtype=jnp.int32)

    @plsc.parallel_loop(0, _ROWS_PER_W, step=_L, unroll=4)
    def chunk(k):
        j = k + lane
        i = base + j
        t = i * (i + 1)
        e = (lax.shift_right_logical(t, jnp.int32(5)) & jnp.int32(768)) | (
            lax.shift_right_logical(t, jnp.int32(6)) & jnp.int32(127))
        c0 = plsc.load_gather(lut_v, [e])
        c1 = plsc.load_gather(lut_v, [e | jnp.int32(128)])
        s = lax.shift_left(t & jnp.int32(0x8000), jnp.int32(16))
        c0 = plsc.bitcast(plsc.bitcast(c0, jnp.int32) ^ s, jnp.float32)
        off = (lax.shift_right_logical(k, jnp.int32(7)) << jnp.int32(8)) | (
            k & jnp.int32(127))
        out_v[pl.ds(off, _L)] = c0
        out_v[pl.ds(off + jnp.int32(128), _L)] = c1

    pltpu.sync_copy(out_v, out_hbm.at[pl.ds(base * 2, 2 * _ROWS_PER_W)])


@jax.jit
def kernel(lut):
    with jax.enable_x64(False):
        mesh = plsc.VectorSubcoreMesh(core_axis_name="c", subcore_axis_name="s")
        run = functools.partial(
            pl.kernel,
            out_type=jax.ShapeDtypeStruct((2 * _N,), jnp.float32),
            mesh=mesh,
            scratch_types=[
                pltpu.VMEM((1024,), jnp.float32),
                pltpu.VMEM((2 * _ROWS_PER_W,), jnp.float32),
                pltpu.SemaphoreType.DMA,
            ],
            compiler_params=pltpu.CompilerParams(needs_layout_passes=False),
        )(_sc_body)
        lut_p = lut.astype(jnp.float32).reshape(4, 128, 2)
        lut_p = lut_p.transpose(0, 2, 1).reshape(1024)
        flat = run(lut_p)
        return flat.reshape(512, 2, 128).transpose(0, 2, 1).reshape(_N, 2)
